# Initial kernel scaffold; baseline (speedup 1.0000x reference)
import jax
import jax.numpy as jnp
from jax.experimental import pallas as pl


def kernel(x1_ids, x2_ids, emb1, emb2, W):
    raise NotImplementedError("write your pallas kernel here")


# speedup vs baseline: 221.1200x; 221.1200x over previous
import functools

import jax
import jax.numpy as jnp
from jax import lax
from jax.experimental import pallas as pl
from jax.experimental.pallas import tpu as pltpu
from jax.experimental.pallas import tpu_sc as plsc

B = 16384
L = 200
OUT = 35

NW = 32
ROWS_PER_W = B // NW
CHUNK_ROWS = 64
NCHUNK = ROWS_PER_W // CHUNK_ROWS
CHUNK_W = CHUNK_ROWS * L
PAIRS = CHUNK_ROWS // 2
OUT_W = ROWS_PER_W * OUT


def _sc_body(x1_hbm, x2_hbm, consts_hbm, out_hbm,
             x1a, x1b, x2a, x2b, out_v, consts_v,
             s1a, s1b, s2a, s2b):
    wid = lax.axis_index("s") * 2 + lax.axis_index("c")
    in_base = wid * (ROWS_PER_W * L)
    out_base = wid * OUT_W

    pltpu.make_async_copy(consts_hbm, consts_v, s1a).start()
    pltpu.make_async_copy(consts_hbm, consts_v, s1a).wait()
    cv = [consts_v[pl.ds(t * 16, 16)] for t in range(15)]

    lane = lax.iota(jnp.int32, 16)
    lo = lane < 8

    x1_bufs, x2_bufs = [x1a, x1b], [x2a, x2b]
    sem1, sem2 = [s1a, s1b], [s2a, s2b]

    def start(c):
        b = c % 2
        off = in_base + c * CHUNK_W
        cp1 = pltpu.make_async_copy(x1_hbm.at[pl.ds(off, CHUNK_W)], x1_bufs[b], sem1[b])
        cp2 = pltpu.make_async_copy(x2_hbm.at[pl.ds(off, CHUNK_W)], x2_bufs[b], sem2[b])
        cp1.start()
        cp2.start()
        return cp1, cp2

    def reduce_pair(ref, base):
        vs = [ref[pl.ds(base + k * 16, 16)] for k in range(25)]
        s0 = vs[0]
        q0 = vs[0] * vs[0]
        for k in range(1, 12):
            s0 = s0 + vs[k]
            q0 = q0 + vs[k] * vs[k]
        w = jnp.where(lo, vs[12], 0)
        s0 = s0 + w
        q0 = q0 + w * w
        w2 = vs[12] - w
        s1 = w2
        q1 = w2 * w2
        for k in range(13, 25):
            s1 = s1 + vs[k]
            q1 = q1 + vs[k] * vs[k]
        return (jnp.sum(s0).astype(jnp.float32), jnp.sum(q0).astype(jnp.float32),
                jnp.sum(s1).astype(jnp.float32), jnp.sum(q1).astype(jnp.float32))

    def emit_row(off, sA, qA, sB, qB):
        for j, o in ((0, 0), (1, 16), (2, 19)):
            vec = (cv[j] + sA * cv[3 + j] + qA * cv[6 + j]
                   + sB * cv[9 + j] + qB * cv[12 + j])
            out_v[pl.ds(off + o, 16)] = vec

    pending = {0: start(0)}
    for c in range(NCHUNK):
        if c + 1 < NCHUNK:
            pending[c + 1] = start(c + 1)
        cp1, cp2 = pending.pop(c)
        cp1.wait()
        cp2.wait()
        xb1, xb2 = x1_bufs[c % 2], x2_bufs[c % 2]
        chunk_out = c * CHUNK_ROWS * OUT

        def pair_body(i, carry):
            base = i * (2 * L)
            sA0, qA0, sA1, qA1 = reduce_pair(xb1, base)
            sB0, qB0, sB1, qB1 = reduce_pair(xb2, base)
            off0 = chunk_out + (2 * i) * OUT
            emit_row(off0, sA0, qA0, sB0, qB0)
            emit_row(off0 + OUT, sA1, qA1, sB1, qB1)
            return carry

        lax.fori_loop(0, PAIRS, pair_body, 0)

    pltpu.make_async_copy(out_v, out_hbm.at[pl.ds(out_base, OUT_W)], s1a).start()
    pltpu.make_async_copy(out_v, out_hbm.at[pl.ds(out_base, OUT_W)], s1a).wait()


@functools.partial(jax.jit, static_argnames=())
def _run(x1f, x2f, consts):
    mesh = plsc.VectorSubcoreMesh(core_axis_name="c", subcore_axis_name="s")
    f = functools.partial(
        pl.kernel,
        mesh=mesh,
        compiler_params=pltpu.CompilerParams(needs_layout_passes=False),
        out_type=jax.ShapeDtypeStruct((B * OUT,), jnp.float32),
        scratch_types=[
            pltpu.VMEM((CHUNK_W,), jnp.int32),
            pltpu.VMEM((CHUNK_W,), jnp.int32),
            pltpu.VMEM((CHUNK_W,), jnp.int32),
            pltpu.VMEM((CHUNK_W,), jnp.int32),
            pltpu.VMEM((OUT_W,), jnp.float32),
            pltpu.VMEM((240,), jnp.float32),
            pltpu.SemaphoreType.DMA,
            pltpu.SemaphoreType.DMA,
            pltpu.SemaphoreType.DMA,
            pltpu.SemaphoreType.DMA,
        ],
    )(_sc_body)
    return f(x1f, x2f, consts)


def kernel(x1_ids, x2_ids, emb1, emb2, W):
    W0, W1 = W[:4], W[4:]

    def coeffs(emb, Wh):
        def dot(v, M):
            return jnp.sum(v[:, None] * M, axis=0)

        a = dot(emb[0], Wh)
        lin = dot(-1.5 * emb[0] + 2.0 * emb[1] - 0.5 * emb[2], Wh) / L
        quad = dot(0.5 * emb[0] - 1.0 * emb[1] + 0.5 * emb[2], Wh) / L
        return a, lin, quad

    a1, l1, k1 = coeffs(emb1.astype(jnp.float32), W0.astype(jnp.float32))
    a2, l2, k2 = coeffs(emb2.astype(jnp.float32), W1.astype(jnp.float32))
    C = a1 + a2

    def slices(v):
        return jnp.stack([v[0:16], v[16:32], v[19:35]])

    consts = jnp.concatenate(
        [slices(C), slices(l1), slices(k1), slices(l2), slices(k2)], axis=0
    ).reshape(-1)

    out = _run(x1_ids.reshape(-1), x2_ids.reshape(-1), consts)
    return out.reshape(B, OUT)


# speedup vs baseline: 321.3725x; 1.4534x over previous
import functools

import jax
import jax.numpy as jnp
from jax import lax
from jax.experimental import pallas as pl
from jax.experimental.pallas import tpu as pltpu
from jax.experimental.pallas import tpu_sc as plsc

B = 16384
L = 200
OUT = 35

NW = 32
ROWS_PER_W = B // NW
CHUNK_ROWS = 64
NCHUNK = ROWS_PER_W // CHUNK_ROWS
ROWS_PER_IT = 2
ITERS = CHUNK_ROWS // ROWS_PER_IT


def _sc_body(x1_hbm, x2_hbm, consts_hbm, out_hbm,
             x1a, x1b, x2a, x2b, outa, outb, consts_v,
             s1a, s1b, s2a, s2b, so_a, so_b):
    wid = lax.axis_index("s") * 2 + lax.axis_index("c")
    row0 = wid * ROWS_PER_W

    pltpu.make_async_copy(consts_hbm, consts_v, s1a).start()
    pltpu.make_async_copy(consts_hbm, consts_v, s1a).wait()
    cv = [consts_v[pl.ds(t * 16, 16)] for t in range(15)]

    lane = lax.iota(jnp.int32, 16)
    hi8 = lane >= 8

    x_bufs = ([x1a, x1b], [x2a, x2b])
    sems = ([s1a, s1b], [s2a, s2b])

    def start(c):
        b = c % 2
        r = row0 + c * CHUNK_ROWS
        cp1 = pltpu.make_async_copy(
            x1_hbm.at[pl.ds(r, CHUNK_ROWS), :], x_bufs[0][b], sems[0][b])
        cp2 = pltpu.make_async_copy(
            x2_hbm.at[pl.ds(r, CHUNK_ROWS), :], x_bufs[1][b], sems[1][b])
        cp1.start()
        cp2.start()
        return cp1, cp2

    def reduce_row(ref, r):
        vs = [ref[r, pl.ds(k * 16, 16)] for k in range(12)]
        tail = ref[r, pl.ds(L - 16, 16)]
        w = jnp.where(hi8, tail, 0)
        s = vs[0] + w
        q = vs[0] * vs[0] + w * w
        for k in range(1, 12):
            s = s + vs[k]
            q = q + vs[k] * vs[k]
        return (jnp.sum(s).astype(jnp.float32), jnp.sum(q).astype(jnp.float32))

    out_bufs = [outa, outb]
    out_sems = [so_a, so_b]

    def emit_row(ob, r, sA, qA, sB, qB):
        for j, o in ((0, 0), (1, 16), (2, 19)):
            vec = (cv[j] + sA * cv[3 + j] + qA * cv[6 + j]
                   + sB * cv[9 + j] + qB * cv[12 + j])
            ob[r, pl.ds(o, 16)] = vec

    pending = {0: start(0)}
    out_pending = {}
    for c in range(NCHUNK):
        if c + 1 < NCHUNK:
            pending[c + 1] = start(c + 1)
        cp1, cp2 = pending.pop(c)
        cp1.wait()
        cp2.wait()
        if c - 2 in out_pending:
            out_pending.pop(c - 2).wait()
        xb1, xb2 = x_bufs[0][c % 2], x_bufs[1][c % 2]
        ob = out_bufs[c % 2]

        def it_body(i, carry):
            for u in range(ROWS_PER_IT):
                r = i * ROWS_PER_IT + u
                sA, qA = reduce_row(xb1, r)
                sB, qB = reduce_row(xb2, r)
                emit_row(ob, r, sA, qA, sB, qB)
            return carry

        lax.fori_loop(0, ITERS, it_body, 0)

        ocp = pltpu.make_async_copy(
            ob, out_hbm.at[pl.ds(row0 + c * CHUNK_ROWS, CHUNK_ROWS), :],
            out_sems[c % 2])
        ocp.start()
        out_pending[c] = ocp
    for c in sorted(out_pending):
        out_pending.pop(c).wait()


@jax.jit
def _run(x1, x2, consts):
    mesh = plsc.VectorSubcoreMesh(core_axis_name="c", subcore_axis_name="s")
    f = functools.partial(
        pl.kernel,
        mesh=mesh,
        compiler_params=pltpu.CompilerParams(needs_layout_passes=False),
        out_type=jax.ShapeDtypeStruct((B, OUT), jnp.float32),
        scratch_types=[
            pltpu.VMEM((CHUNK_ROWS, L), jnp.int32),
            pltpu.VMEM((CHUNK_ROWS, L), jnp.int32),
            pltpu.VMEM((CHUNK_ROWS, L), jnp.int32),
            pltpu.VMEM((CHUNK_ROWS, L), jnp.int32),
            pltpu.VMEM((CHUNK_ROWS, OUT), jnp.float32),
            pltpu.VMEM((CHUNK_ROWS, OUT), jnp.float32),
            pltpu.VMEM((240,), jnp.float32),
            pltpu.SemaphoreType.DMA,
            pltpu.SemaphoreType.DMA,
            pltpu.SemaphoreType.DMA,
            pltpu.SemaphoreType.DMA,
            pltpu.SemaphoreType.DMA,
            pltpu.SemaphoreType.DMA,
        ],
    )(_sc_body)
    return f(x1, x2, consts)


def kernel(x1_ids, x2_ids, emb1, emb2, W):
    W0, W1 = W[:4], W[4:]

    def coeffs(emb, Wh):
        def dot(v, M):
            return jnp.sum(v[:, None] * M, axis=0)

        a = dot(emb[0], Wh)
        lin = dot(-1.5 * emb[0] + 2.0 * emb[1] - 0.5 * emb[2], Wh) / L
        quad = dot(0.5 * emb[0] - 1.0 * emb[1] + 0.5 * emb[2], Wh) / L
        return a, lin, quad

    a1, l1, k1 = coeffs(emb1.astype(jnp.float32), W0.astype(jnp.float32))
    a2, l2, k2 = coeffs(emb2.astype(jnp.float32), W1.astype(jnp.float32))
    C = a1 + a2

    def slices(v):
        return jnp.stack([v[0:16], v[16:32], v[19:35]])

    consts = jnp.concatenate(
        [slices(C), slices(l1), slices(k1), slices(l2), slices(k2)], axis=0
    ).reshape(-1)

    return _run(x1_ids, x2_ids, consts)


# speedup vs baseline: 321.4396x; 1.0002x over previous
import functools

import jax
import jax.numpy as jnp
from jax import lax
from jax.experimental import pallas as pl
from jax.experimental.pallas import tpu as pltpu
from jax.experimental.pallas import tpu_sc as plsc

B = 16384
L = 200
OUT = 35

NW = 32
ROWS_PER_W = B // NW
CHUNK_ROWS = 64
NCHUNK = ROWS_PER_W // CHUNK_ROWS
ROWS_PER_IT = 2
ITERS = CHUNK_ROWS // ROWS_PER_IT


def _sc_body(x1_hbm, x2_hbm, consts_hbm, out_hbm,
             x1a, x1b, x2a, x2b, outa, outb, consts_v,
             s1a, s1b, s2a, s2b, so_a, so_b):
    wid = lax.axis_index("s") * 2 + lax.axis_index("c")
    row0 = wid * ROWS_PER_W

    pltpu.make_async_copy(consts_hbm, consts_v, s1a).start()
    pltpu.make_async_copy(consts_hbm, consts_v, s1a).wait()
    cv = [consts_v[pl.ds(t * 16, 16)] for t in range(15)]

    lane = lax.iota(jnp.int32, 16)
    hi8 = lane >= 8

    x_bufs = ([x1a, x1b], [x2a, x2b])
    sems = ([s1a, s1b], [s2a, s2b])

    def start(c):
        b = c % 2
        r = row0 + c * CHUNK_ROWS
        cp1 = pltpu.make_async_copy(
            x1_hbm.at[pl.ds(r, CHUNK_ROWS), :], x_bufs[0][b], sems[0][b])
        cp2 = pltpu.make_async_copy(
            x2_hbm.at[pl.ds(r, CHUNK_ROWS), :], x_bufs[1][b], sems[1][b])
        cp1.start()
        cp2.start()
        return cp1, cp2

    def reduce_row(ref, r):
        vs = [ref[r, pl.ds(k * 16, 16)] for k in range(12)]
        tail = ref[r, pl.ds(L - 16, 16)]
        w = jnp.where(hi8, tail, 0)
        s = vs[0] + w
        q = vs[0] * vs[0] + w * w
        for k in range(1, 12):
            s = s + vs[k]
            q = q + vs[k] * vs[k]
        return (jnp.sum(s).astype(jnp.float32), jnp.sum(q).astype(jnp.float32))

    out_bufs = [outa, outb]
    out_sems = [so_a, so_b]

    def emit_row(ob, r, sA, qA, sB, qB):
        for j, o in ((0, 0), (1, 16), (2, 19)):
            vec = (cv[j] + sA * cv[3 + j] + qA * cv[6 + j]
                   + sB * cv[9 + j] + qB * cv[12 + j])
            ob[r, pl.ds(o, 16)] = vec

    pending = {0: start(0)}
    out_pending = {}
    for c in range(NCHUNK):
        if c + 1 < NCHUNK:
            pending[c + 1] = start(c + 1)
        cp1, cp2 = pending.pop(c)
        cp1.wait()
        cp2.wait()
        if c - 2 in out_pending:
            out_pending.pop(c - 2).wait()
        xb1, xb2 = x_bufs[0][c % 2], x_bufs[1][c % 2]
        ob = out_bufs[c % 2]

        def it_body(i, carry):
            for u in range(ROWS_PER_IT):
                r = i * ROWS_PER_IT + u
                sA, qA = reduce_row(xb1, r)
                sB, qB = reduce_row(xb2, r)
                emit_row(ob, r, sA, qA, sB, qB)
            return carry

        lax.fori_loop(0, ITERS, it_body, 0)

        ocp = pltpu.make_async_copy(
            ob, out_hbm.at[pl.ds(row0 + c * CHUNK_ROWS, CHUNK_ROWS), :],
            out_sems[c % 2])
        ocp.start()
        out_pending[c] = ocp
    for c in sorted(out_pending):
        out_pending.pop(c).wait()


@jax.jit
def _run(x1, x2, consts):
    mesh = plsc.VectorSubcoreMesh(core_axis_name="c", subcore_axis_name="s")
    f = functools.partial(
        pl.kernel,
        mesh=mesh,
        compiler_params=pltpu.CompilerParams(
            needs_layout_passes=False, use_tc_tiling_on_sc=True),
        out_type=jax.ShapeDtypeStruct((B, OUT), jnp.float32),
        scratch_types=[
            pltpu.VMEM((CHUNK_ROWS, L), jnp.int32),
            pltpu.VMEM((CHUNK_ROWS, L), jnp.int32),
            pltpu.VMEM((CHUNK_ROWS, L), jnp.int32),
            pltpu.VMEM((CHUNK_ROWS, L), jnp.int32),
            pltpu.VMEM((CHUNK_ROWS, OUT), jnp.float32),
            pltpu.VMEM((CHUNK_ROWS, OUT), jnp.float32),
            pltpu.VMEM((240,), jnp.float32),
            pltpu.SemaphoreType.DMA,
            pltpu.SemaphoreType.DMA,
            pltpu.SemaphoreType.DMA,
            pltpu.SemaphoreType.DMA,
            pltpu.SemaphoreType.DMA,
            pltpu.SemaphoreType.DMA,
        ],
    )(_sc_body)
    return f(x1, x2, consts)


def kernel(x1_ids, x2_ids, emb1, emb2, W):
    W0, W1 = W[:4], W[4:]

    def coeffs(emb, Wh):
        def dot(v, M):
            return jnp.sum(v[:, None] * M, axis=0)

        a = dot(emb[0], Wh)
        lin = dot(-1.5 * emb[0] + 2.0 * emb[1] - 0.5 * emb[2], Wh) / L
        quad = dot(0.5 * emb[0] - 1.0 * emb[1] + 0.5 * emb[2], Wh) / L
        return a, lin, quad

    a1, l1, k1 = coeffs(emb1.astype(jnp.float32), W0.astype(jnp.float32))
    a2, l2, k2 = coeffs(emb2.astype(jnp.float32), W1.astype(jnp.float32))
    C = a1 + a2

    def slices(v):
        return jnp.stack([v[0:16], v[16:32], v[19:35]])

    consts = jnp.concatenate(
        [slices(C), slices(l1), slices(k1), slices(l2), slices(k2)], axis=0
    ).reshape(-1)

    return _run(x1_ids, x2_ids, consts)


# speedup vs baseline: 620.3291x; 1.9298x over previous
import functools

import jax
import jax.numpy as jnp
from jax import lax
from jax.experimental import pallas as pl
from jax.experimental.pallas import tpu as pltpu
from jax.experimental.pallas import tpu_sc as plsc

B = 16384
L = 200
OUT = 35

NW = 32
COLS_PER_W = B // NW
STAGE_ROWS = 40
NSTAGE = L // STAGE_ROWS
NPAIR = COLS_PER_W // 32

_ILV = plsc.PackFormat.INTERLEAVED


def _sc_body(x1_hbm, x2_hbm, consts_hbm, out_hbm,
             x1h0, x1h1, x2h0, x2h1, out_v, consts_v, mom_v,
             s1a, s1b, s2a, s2b, so):
    wid = lax.axis_index("s") * 2 + lax.axis_index("c")
    col0 = wid * COLS_PER_W

    pltpu.make_async_copy(consts_hbm, consts_v, so).start()
    pltpu.make_async_copy(consts_hbm, consts_v, so).wait()

    x_bufs = ([x1h0, x1h1], [x2h0, x2h1])
    sems = ([s1a, s1b], [s2a, s2b])

    def start(s):
        b = s % 2
        rows = s * STAGE_ROWS
        cp1 = pltpu.make_async_copy(
            x1_hbm.at[pl.ds(rows, STAGE_ROWS), pl.ds(col0, COLS_PER_W)],
            x_bufs[0][b], sems[0][b])
        cp2 = pltpu.make_async_copy(
            x2_hbm.at[pl.ds(rows, STAGE_ROWS), pl.ds(col0, COLS_PER_W)],
            x_bufs[1][b], sems[1][b])
        cp1.start()
        cp2.start()
        return cp1, cp2

    def accum(ref, offA, offB):
        pp = plsc.pack(ref[0, pl.ds(offA, 16)], ref[0, pl.ds(offB, 16)],
                       format=_ILV)
        s16 = pp
        q16 = pp * pp
        for r in range(1, STAGE_ROWS):
            pp = plsc.pack(ref[r, pl.ds(offA, 16)], ref[r, pl.ds(offB, 16)],
                           format=_ILV)
            s16 = s16 + pp
            q16 = q16 + pp * pp
        return s16, q16

    def mom_slot(k, p):
        return mom_v[k, pl.ds(p * 16, 16)]

    pending = {0: start(0)}
    for s in range(NSTAGE):
        if s + 1 < NSTAGE:
            pending[s + 1] = start(s + 1)
        cp1, cp2 = pending.pop(s)
        cp1.wait()
        cp2.wait()
        xb1, xb2 = x_bufs[0][s % 2], x_bufs[1][s % 2]

        if s == 0:
            def body0(p, carry):
                offA = p * 32
                s1, t1 = accum(xb1, offA, offA + 16)
                s2, t2 = accum(xb2, offA, offA + 16)
                for k, v in enumerate((s1, t1, s2, t2)):
                    mom_v[0 + k, pl.ds(p * 16, 16)] = plsc.bitcast(v, jnp.int32)
                return carry
            lax.fori_loop(0, NPAIR, body0, 0)
        elif s < NSTAGE - 1:
            def body_mid(p, carry):
                offA = p * 32
                s1, t1 = accum(xb1, offA, offA + 16)
                s2, t2 = accum(xb2, offA, offA + 16)
                for k, v in enumerate((s1, t1, s2, t2)):
                    acc = plsc.bitcast(mom_slot(k, p), jnp.int16)
                    mom_v[k, pl.ds(p * 16, 16)] = plsc.bitcast(v + acc, jnp.int32)
                return carry
            lax.fori_loop(0, NPAIR, body_mid, 0)
        else:
            def body_fin(p, carry):
                offA = p * 32
                offB = offA + 16
                s1, t1 = accum(xb1, offA, offB)
                s2, t2 = accum(xb2, offA, offB)
                tot = [v + plsc.bitcast(mom_slot(k, p), jnp.int16)
                       for k, v in enumerate((s1, t1, s2, t2))]
                fA, fB = [], []
                for k in range(4):
                    va, vb = plsc.unpack(tot[k], format=_ILV)
                    fA.append(va.astype(jnp.float32))
                    fB.append(vb.astype(jnp.float32))
                for j in range(OUT):
                    cb = [consts_v[pl.ds((k * OUT + j) * 16, 16)]
                          for k in range(5)]
                    out_v[j, pl.ds(offA, 16)] = (
                        cb[0] + fA[0] * cb[1] + fA[1] * cb[2]
                        + fA[2] * cb[3] + fA[3] * cb[4])
                    out_v[j, pl.ds(offB, 16)] = (
                        cb[0] + fB[0] * cb[1] + fB[1] * cb[2]
                        + fB[2] * cb[3] + fB[3] * cb[4])
                return carry
            lax.fori_loop(0, NPAIR, body_fin, 0)

    ocp = pltpu.make_async_copy(out_v, out_hbm.at[:, pl.ds(col0, COLS_PER_W)], so)
    ocp.start()
    ocp.wait()


@jax.jit
def _run(x1t, x2t, consts):
    mesh = plsc.VectorSubcoreMesh(core_axis_name="c", subcore_axis_name="s")
    f = functools.partial(
        pl.kernel,
        mesh=mesh,
        compiler_params=pltpu.CompilerParams(needs_layout_passes=False),
        out_type=jax.ShapeDtypeStruct((OUT, B), jnp.float32),
        scratch_types=[
            pltpu.VMEM((STAGE_ROWS, COLS_PER_W), jnp.int32),
            pltpu.VMEM((STAGE_ROWS, COLS_PER_W), jnp.int32),
            pltpu.VMEM((STAGE_ROWS, COLS_PER_W), jnp.int32),
            pltpu.VMEM((STAGE_ROWS, COLS_PER_W), jnp.int32),
            pltpu.VMEM((OUT, COLS_PER_W), jnp.float32),
            pltpu.VMEM((5 * OUT * 16,), jnp.float32),
            pltpu.VMEM((4, COLS_PER_W // 2), jnp.int32),
            pltpu.SemaphoreType.DMA,
            pltpu.SemaphoreType.DMA,
            pltpu.SemaphoreType.DMA,
            pltpu.SemaphoreType.DMA,
            pltpu.SemaphoreType.DMA,
        ],
    )(_sc_body)
    return f(x1t, x2t, consts)


def kernel(x1_ids, x2_ids, emb1, emb2, W):
    W0, W1 = W[:4], W[4:]

    def coeffs(emb, Wh):
        def dot(v, M):
            return jnp.sum(v[:, None] * M, axis=0)

        a = dot(emb[0], Wh)
        lin = dot(-1.5 * emb[0] + 2.0 * emb[1] - 0.5 * emb[2], Wh) / L
        quad = dot(0.5 * emb[0] - 1.0 * emb[1] + 0.5 * emb[2], Wh) / L
        return a, lin, quad

    a1, l1, k1 = coeffs(emb1.astype(jnp.float32), W0.astype(jnp.float32))
    a2, l2, k2 = coeffs(emb2.astype(jnp.float32), W1.astype(jnp.float32))
    consts = jnp.stack([a1 + a2, l1, k1, l2, k2])
    consts_b = jnp.broadcast_to(consts[:, :, None], (5, OUT, 16)).reshape(-1)

    out_t = _run(x1_ids.T, x2_ids.T, consts_b)
    return out_t.T


# speedup vs baseline: 626.6082x; 1.0101x over previous
import functools

import jax
import jax.numpy as jnp
from jax import lax
from jax.experimental import pallas as pl
from jax.experimental.pallas import tpu as pltpu
from jax.experimental.pallas import tpu_sc as plsc

B = 16384
L = 200
OUT = 35

NW = 32
COLS_PER_W = B // NW
STAGE_ROWS = 40
NSTAGE = L // STAGE_ROWS
NPAIR = COLS_PER_W // 32

_ILV = plsc.PackFormat.INTERLEAVED


def _sc_body(x1_hbm, x2_hbm, consts_hbm, out_hbm,
             x1h0, x1h1, x2h0, x2h1, out_v, consts_v, mom_v,
             s1a, s1b, s2a, s2b, so):
    wid = lax.axis_index("s") * 2 + lax.axis_index("c")
    col0 = wid * COLS_PER_W

    pltpu.make_async_copy(consts_hbm, consts_v, so).start()
    pltpu.make_async_copy(consts_hbm, consts_v, so).wait()

    x_bufs = ([x1h0, x1h1], [x2h0, x2h1])
    sems = ([s1a, s1b], [s2a, s2b])

    def start(s):
        b = s % 2
        rows = s * STAGE_ROWS
        cp1 = pltpu.make_async_copy(
            x1_hbm.at[pl.ds(rows, STAGE_ROWS), pl.ds(col0, COLS_PER_W)],
            x_bufs[0][b], sems[0][b])
        cp2 = pltpu.make_async_copy(
            x2_hbm.at[pl.ds(rows, STAGE_ROWS), pl.ds(col0, COLS_PER_W)],
            x_bufs[1][b], sems[1][b])
        cp1.start()
        cp2.start()
        return cp1, cp2

    def accum(ref, offA, offB):
        pp = plsc.pack(ref[0, pl.ds(offA, 16)], ref[0, pl.ds(offB, 16)],
                       format=_ILV)
        s16 = pp
        q16 = pp * pp
        for r in range(1, STAGE_ROWS):
            pp = plsc.pack(ref[r, pl.ds(offA, 16)], ref[r, pl.ds(offB, 16)],
                           format=_ILV)
            s16 = s16 + pp
            q16 = q16 + pp * pp
        return s16, q16

    def mom_slot(k, p):
        return mom_v[k, pl.ds(p * 16, 16)]

    pending = {0: start(0)}
    for s in range(NSTAGE):
        if s + 1 < NSTAGE:
            pending[s + 1] = start(s + 1)
        cp1, cp2 = pending.pop(s)
        cp1.wait()
        cp2.wait()
        xb1, xb2 = x_bufs[0][s % 2], x_bufs[1][s % 2]

        if s == 0:
            def body0(p, carry):
                offA = p * 32
                s1, q1 = accum(xb1, offA, offA + 16)
                s2, q2 = accum(xb2, offA, offA + 16)
                for k, v in enumerate((s1, q1, s2, q2)):
                    mom_v[k, pl.ds(p * 16, 16)] = plsc.bitcast(v, jnp.int32)
                return carry
            lax.fori_loop(0, NPAIR, body0, 0)
        else:
            def body_mid(p, carry):
                offA = p * 32
                s1, q1 = accum(xb1, offA, offA + 16)
                s2, q2 = accum(xb2, offA, offA + 16)
                for k, v in enumerate((s1, q1, s2, q2)):
                    acc = plsc.bitcast(mom_slot(k, p), jnp.int16)
                    mom_v[k, pl.ds(p * 16, 16)] = plsc.bitcast(v + acc, jnp.int32)
                return carry
            lax.fori_loop(0, NPAIR, body_mid, 0)

    def body_emit(p, carry):
        offA = p * 32
        offB = offA + 16
        fA, fB = [], []
        for k in range(4):
            va, vb = plsc.unpack(plsc.bitcast(mom_slot(k, p), jnp.int16),
                                 format=_ILV)
            fA.append(va.astype(jnp.float32))
            fB.append(vb.astype(jnp.float32))
        for j in range(OUT):
            cb = [consts_v[pl.ds((k * OUT + j) * 16, 16)] for k in range(5)]
            out_v[j, pl.ds(offA, 16)] = (
                (cb[0] + fA[0] * cb[1]) + (fA[1] * cb[2] + fA[2] * cb[3])
                + fA[3] * cb[4])
            out_v[j, pl.ds(offB, 16)] = (
                (cb[0] + fB[0] * cb[1]) + (fB[1] * cb[2] + fB[2] * cb[3])
                + fB[3] * cb[4])
        return carry

    lax.fori_loop(0, NPAIR, body_emit, 0)

    ocp = pltpu.make_async_copy(out_v, out_hbm.at[:, pl.ds(col0, COLS_PER_W)], so)
    ocp.start()
    ocp.wait()


@jax.jit
def _run(x1t, x2t, consts):
    mesh = plsc.VectorSubcoreMesh(core_axis_name="c", subcore_axis_name="s")
    f = functools.partial(
        pl.kernel,
        mesh=mesh,
        compiler_params=pltpu.CompilerParams(needs_layout_passes=False),
        out_type=jax.ShapeDtypeStruct((OUT, B), jnp.float32),
        scratch_types=[
            pltpu.VMEM((STAGE_ROWS, COLS_PER_W), jnp.int32),
            pltpu.VMEM((STAGE_ROWS, COLS_PER_W), jnp.int32),
            pltpu.VMEM((STAGE_ROWS, COLS_PER_W), jnp.int32),
            pltpu.VMEM((STAGE_ROWS, COLS_PER_W), jnp.int32),
            pltpu.VMEM((OUT, COLS_PER_W), jnp.float32),
            pltpu.VMEM((5 * OUT * 16,), jnp.float32),
            pltpu.VMEM((4, COLS_PER_W // 2), jnp.int32),
            pltpu.SemaphoreType.DMA,
            pltpu.SemaphoreType.DMA,
            pltpu.SemaphoreType.DMA,
            pltpu.SemaphoreType.DMA,
            pltpu.SemaphoreType.DMA,
        ],
    )(_sc_body)
    return f(x1t, x2t, consts)


def kernel(x1_ids, x2_ids, emb1, emb2, W):
    e1 = emb1.astype(jnp.float32)
    e2 = emb2.astype(jnp.float32)
    z = jnp.zeros((4,), jnp.float32)
    lin1 = (-1.5 * e1[0] + 2.0 * e1[1] - 0.5 * e1[2]) / L
    quad1 = (0.5 * e1[0] - 1.0 * e1[1] + 0.5 * e1[2]) / L
    lin2 = (-1.5 * e2[0] + 2.0 * e2[1] - 0.5 * e2[2]) / L
    quad2 = (0.5 * e2[0] - 1.0 * e2[1] + 0.5 * e2[2]) / L
    G = jnp.stack([
        jnp.concatenate([e1[0], e2[0]]),
        jnp.concatenate([lin1, z]),
        jnp.concatenate([quad1, z]),
        jnp.concatenate([z, lin2]),
        jnp.concatenate([z, quad2]),
    ])
    consts = jax.lax.dot(G, W.astype(jnp.float32),
                         precision=jax.lax.Precision.HIGHEST)
    consts_b = jnp.broadcast_to(consts[:, :, None], (5, OUT, 16)).reshape(-1)

    out_t = _run(x1_ids.T, x2_ids.T, consts_b)
    return out_t.T


# speedup vs baseline: 671.6695x; 1.0719x over previous
import functools

import jax
import jax.numpy as jnp
from jax import lax
from jax.experimental import pallas as pl
from jax.experimental.pallas import tpu as pltpu
from jax.experimental.pallas import tpu_sc as plsc

B = 16384
L = 200
OUT = 35

NW = 32
COLS_PER_W = B // NW
STAGE_ROWS = 40
NSTAGE = L // STAGE_ROWS
NPAIR = COLS_PER_W // 32

_ILV = plsc.PackFormat.INTERLEAVED


def _sc_body(x1_hbm, x2_hbm, consts_hbm, out_hbm,
             x1h0, x1h1, x2h0, x2h1, out_v, consts_v, mom_v,
             s1a, s1b, s2a, s2b, so):
    wid = lax.axis_index("s") * 2 + lax.axis_index("c")
    col0 = wid * COLS_PER_W

    pltpu.make_async_copy(consts_hbm, consts_v, so).start()
    pltpu.make_async_copy(consts_hbm, consts_v, so).wait()

    x_bufs = ([x1h0, x1h1], [x2h0, x2h1])
    sems = ([s1a, s1b], [s2a, s2b])

    def start(s):
        b = s % 2
        rows = s * STAGE_ROWS
        cp1 = pltpu.make_async_copy(
            x1_hbm.at[pl.ds(rows, STAGE_ROWS), pl.ds(col0, COLS_PER_W)],
            x_bufs[0][b], sems[0][b])
        cp2 = pltpu.make_async_copy(
            x2_hbm.at[pl.ds(rows, STAGE_ROWS), pl.ds(col0, COLS_PER_W)],
            x_bufs[1][b], sems[1][b])
        cp1.start()
        cp2.start()
        return cp1, cp2

    def accum(ref, offA, offB):
        pp = plsc.pack(ref[0, pl.ds(offA, 16)], ref[0, pl.ds(offB, 16)],
                       format=_ILV)
        s16 = pp
        q16 = pp * pp
        for r in range(1, STAGE_ROWS):
            pp = plsc.pack(ref[r, pl.ds(offA, 16)], ref[r, pl.ds(offB, 16)],
                           format=_ILV)
            s16 = s16 + pp
            q16 = q16 + pp * pp
        return s16, q16

    def mom_slot(k, p):
        return mom_v[k, pl.ds(p * 16, 16)]

    pending = {0: start(0)}
    for s in range(NSTAGE):
        if s + 1 < NSTAGE:
            pending[s + 1] = start(s + 1)
        cp1, cp2 = pending.pop(s)
        cp1.wait()
        cp2.wait()
        xb1, xb2 = x_bufs[0][s % 2], x_bufs[1][s % 2]

        if s == 0:
            def body0(p, carry):
                offA = p * 32
                s1, q1 = accum(xb1, offA, offA + 16)
                s2, q2 = accum(xb2, offA, offA + 16)
                for k, v in enumerate((s1, q1, s2, q2)):
                    mom_v[k, pl.ds(p * 16, 16)] = plsc.bitcast(v, jnp.int32)
                return carry
            lax.fori_loop(0, NPAIR, body0, 0)
        else:
            def body_mid(p, carry):
                offA = p * 32
                s1, q1 = accum(xb1, offA, offA + 16)
                s2, q2 = accum(xb2, offA, offA + 16)
                for k, v in enumerate((s1, q1, s2, q2)):
                    acc = plsc.bitcast(mom_slot(k, p), jnp.int16)
                    mom_v[k, pl.ds(p * 16, 16)] = plsc.bitcast(v + acc, jnp.int32)
                return carry
            lax.fori_loop(0, NPAIR, body_mid, 0)

    def body_emit(p, carry):
        offA = p * 32
        offB = offA + 16
        fA, fB = [], []
        for k in range(4):
            va, vb = plsc.unpack(plsc.bitcast(mom_slot(k, p), jnp.int16),
                                 format=_ILV)
            fA.append(va.astype(jnp.float32))
            fB.append(vb.astype(jnp.float32))
        def load_cb(j):
            return [consts_v[pl.ds((k * OUT + j) * 16, 16)] for k in range(5)]

        cb = load_cb(0)
        for j in range(OUT):
            nxt = load_cb(j + 1) if j + 1 < OUT else None
            out_v[j, pl.ds(offA, 16)] = (
                (cb[0] + fA[0] * cb[1]) + (fA[1] * cb[2] + fA[2] * cb[3])
                + fA[3] * cb[4])
            out_v[j, pl.ds(offB, 16)] = (
                (cb[0] + fB[0] * cb[1]) + (fB[1] * cb[2] + fB[2] * cb[3])
                + fB[3] * cb[4])
            cb = nxt
        return carry

    lax.fori_loop(0, NPAIR, body_emit, 0)

    ocp = pltpu.make_async_copy(out_v, out_hbm.at[:, pl.ds(col0, COLS_PER_W)], so)
    ocp.start()
    ocp.wait()


@jax.jit
def _run(x1t, x2t, consts):
    mesh = plsc.VectorSubcoreMesh(core_axis_name="c", subcore_axis_name="s")
    f = functools.partial(
        pl.kernel,
        mesh=mesh,
        compiler_params=pltpu.CompilerParams(needs_layout_passes=False),
        out_type=jax.ShapeDtypeStruct((OUT, B), jnp.float32),
        scratch_types=[
            pltpu.VMEM((STAGE_ROWS, COLS_PER_W), jnp.int32),
            pltpu.VMEM((STAGE_ROWS, COLS_PER_W), jnp.int32),
            pltpu.VMEM((STAGE_ROWS, COLS_PER_W), jnp.int32),
            pltpu.VMEM((STAGE_ROWS, COLS_PER_W), jnp.int32),
            pltpu.VMEM((OUT, COLS_PER_W), jnp.float32),
            pltpu.VMEM((5 * OUT * 16,), jnp.float32),
            pltpu.VMEM((4, COLS_PER_W // 2), jnp.int32),
            pltpu.SemaphoreType.DMA,
            pltpu.SemaphoreType.DMA,
            pltpu.SemaphoreType.DMA,
            pltpu.SemaphoreType.DMA,
            pltpu.SemaphoreType.DMA,
        ],
    )(_sc_body)
    return f(x1t, x2t, consts)


def kernel(x1_ids, x2_ids, emb1, emb2, W):
    e1 = emb1.astype(jnp.float32)
    e2 = emb2.astype(jnp.float32)
    z = jnp.zeros((4,), jnp.float32)
    lin1 = (-1.5 * e1[0] + 2.0 * e1[1] - 0.5 * e1[2]) / L
    quad1 = (0.5 * e1[0] - 1.0 * e1[1] + 0.5 * e1[2]) / L
    lin2 = (-1.5 * e2[0] + 2.0 * e2[1] - 0.5 * e2[2]) / L
    quad2 = (0.5 * e2[0] - 1.0 * e2[1] + 0.5 * e2[2]) / L
    G = jnp.stack([
        jnp.concatenate([e1[0], e2[0]]),
        jnp.concatenate([lin1, z]),
        jnp.concatenate([quad1, z]),
        jnp.concatenate([z, lin2]),
        jnp.concatenate([z, quad2]),
    ])
    consts = jax.lax.dot(G, W.astype(jnp.float32),
                         precision=jax.lax.Precision.HIGHEST)
    consts_b = jnp.broadcast_to(consts[:, :, None], (5, OUT, 16)).reshape(-1)

    out_t = _run(x1_ids.T, x2_ids.T, consts_b)
    return out_t.T
